# baseline (speedup 1.0000x reference)
import functools

import jax
import jax.numpy as jnp
from jax import lax
from jax.experimental import pallas as pl
from jax.experimental.pallas import tpu as pltpu
from jax.experimental.pallas import tpu_sc as plsc

N = 10000
E = 320000
D = 128
G = 64

B = 128
NW = 32
EPAD = 323584
CH_A = EPAD // NW // B
CH_D = EPAD // 16 // B
NDEG = 10240
RPT = NDEG // 16
NAGG = 10112
RPA = NAGG // 16
IDEP = 4

_mesh = plsc.VectorSubcoreMesh(core_axis_name="c", subcore_axis_name="s")


@functools.partial(
    pl.kernel,
    out_type=jax.ShapeDtypeStruct((2, NDEG), jnp.float32),
    mesh=_mesh,
    scratch_types=[
        pltpu.VMEM((CH_D, B), jnp.int32),
        pltpu.VMEM((B,), jnp.float32),
        pltpu.VMEM((RPT,), jnp.float32),
        pltpu.VMEM_SHARED((NDEG,), jnp.float32),
    ],
)
def _deg_kernel(edges_hbm, zeros_hbm, out_hbm, idx_v, ones_v, zb_v, acc_sh):
    c = lax.axis_index("c")
    s = lax.axis_index("s")
    pltpu.sync_copy(zeros_hbm, zb_v)
    pltpu.sync_copy(zb_v, acc_sh.at[pl.ds(s * RPT, RPT)])
    for i in range(B // 16):
        ones_v[pl.ds(i * 16, 16)] = jnp.full((16,), 1.0, jnp.float32)
    pltpu.sync_copy(edges_hbm.at[c, s], idx_v)
    plsc.subcore_barrier()
    for j in range(CH_D):
        pltpu.sync_copy(ones_v, acc_sh.at[idx_v.at[j]], add=True)
    plsc.subcore_barrier()
    pltpu.sync_copy(acc_sh.at[pl.ds(s * RPT, RPT)], zb_v)
    pltpu.sync_copy(zb_v, out_hbm.at[c, pl.ds(s * RPT, RPT)])


@functools.partial(
    pl.kernel,
    out_type=jax.ShapeDtypeStruct((2, NAGG, D), jnp.float32),
    mesh=_mesh,
    scratch_types=[
        pltpu.VMEM((IDEP, 2, B), jnp.int32),
        pltpu.VMEM((B, D), jnp.float32),
        pltpu.VMEM((B, D), jnp.float32),
        pltpu.VMEM((B, D), jnp.float32),
        pltpu.VMEM_SHARED((NAGG, D), jnp.float32),
        pltpu.SemaphoreType.DMA,
        pltpu.SemaphoreType.DMA,
        pltpu.SemaphoreType.DMA,
        pltpu.SemaphoreType.DMA,
        pltpu.SemaphoreType.DMA,
        pltpu.SemaphoreType.DMA,
        pltpu.SemaphoreType.DMA,
        pltpu.SemaphoreType.DMA,
        pltpu.SemaphoreType.DMA,
    ],
)
def _agg_kernel(h_hbm, edges_hbm, zeros_hbm, out_hbm,
                idx_v, r0, r1, r2, acc_sh,
                g0, g1, g2, s0, s1, i0, i1, i2, i3):
    c = lax.axis_index("c")
    s = lax.axis_index("s")
    w = c * 16 + s
    rows = (r0, r1, r2)
    gsems = (g0, g1, g2)
    ssems = (s0, s1)
    isems = (i0, i1, i2, i3)
    pltpu.sync_copy(zeros_hbm, r0)
    for t in range(5):
        n = 128 if t < 4 else RPA - 512
        pltpu.sync_copy(r0.at[pl.ds(0, n)],
                        acc_sh.at[pl.ds(s * RPA + t * 128, n)])
    plsc.subcore_barrier()
    gp = [None] * CH_A
    ip = [None] * CH_A
    sp = [None] * CH_A
    for k in range(min(3, CH_A)):
        pltpu.sync_copy(edges_hbm.at[w, k], idx_v.at[k])
    for k in range(min(2, CH_A)):
        gp[k] = pltpu.async_copy(h_hbm.at[idx_v.at[k, 0]], rows[k], gsems[k])
    for j in range(CH_A):
        gp[j].wait()
        sp[j] = pltpu.async_copy(rows[j % 3], acc_sh.at[idx_v.at[j % 4, 1]],
                                 ssems[j % 2], add=True)
        if j >= 1:
            sp[j - 1].wait()
        if j + 3 < CH_A:
            ip[j + 3] = pltpu.async_copy(edges_hbm.at[w, j + 3],
                                         idx_v.at[(j + 3) % 4],
                                         isems[(j + 3) % 4])
        if j + 2 < CH_A:
            if ip[j + 2] is not None:
                ip[j + 2].wait()
            gp[j + 2] = pltpu.async_copy(h_hbm.at[idx_v.at[(j + 2) % 4, 0]],
                                         rows[(j + 2) % 3],
                                         gsems[(j + 2) % 3])
    sp[CH_A - 1].wait()
    plsc.subcore_barrier()
    for t in range(5):
        n = 128 if t < 4 else RPA - 512
        pltpu.sync_copy(acc_sh.at[pl.ds(s * RPA + t * 128, n)],
                        r0.at[pl.ds(0, n)])
        pltpu.sync_copy(r0.at[pl.ds(0, n)],
                        out_hbm.at[c, pl.ds(s * RPA + t * 128, n)])


def _k1_body(x_ref, w_ref, deg_ref, o_ref):
    norm = lax.rsqrt(jnp.maximum(deg_ref[...], 1.0))
    o_ref[...] = jnp.dot(x_ref[...], w_ref[...],
                         preferred_element_type=jnp.float32) * norm


_k1 = pl.pallas_call(
    _k1_body,
    out_shape=jax.ShapeDtypeStruct((N, D), jnp.float32),
)


def _k2_body(p_ref, indeg_ref, outdeg_ref, b1_ref, g1_ref, be1_ref, w2_ref,
             o_ref):
    p = p_ref[0, :N, :] + p_ref[1, :N, :]
    z = p * lax.rsqrt(jnp.maximum(indeg_ref[...], 1.0)) + b1_ref[...]
    mu = jnp.mean(z, axis=0, keepdims=True)
    var = jnp.mean((z - mu) ** 2, axis=0, keepdims=True)
    zn = (z - mu) * lax.rsqrt(var + 1e-5) * g1_ref[...] + be1_ref[...]
    zn = jnp.maximum(zn, 0.0)
    nsrc = lax.rsqrt(jnp.maximum(outdeg_ref[...], 1.0))
    o_ref[...] = jnp.dot(zn, w2_ref[...],
                         preferred_element_type=jnp.float32) * nsrc


_k2 = pl.pallas_call(
    _k2_body,
    out_shape=jax.ShapeDtypeStruct((N, D), jnp.float32),
)


def _k3_body(p_ref, indeg_ref, b2_ref, g2_ref, be2_ref, gid_ref, o_ref):
    p = p_ref[0, :N, :] + p_ref[1, :N, :]
    z = p * lax.rsqrt(jnp.maximum(indeg_ref[...], 1.0)) + b2_ref[...]
    mu = jnp.mean(z, axis=0, keepdims=True)
    var = jnp.mean((z - mu) ** 2, axis=0, keepdims=True)
    zn = (z - mu) * lax.rsqrt(var + 1e-5) * g2_ref[...] + be2_ref[...]
    iota = lax.broadcasted_iota(jnp.int32, (1, G), 1)
    onehot = (gid_ref[...] == iota).astype(jnp.float32)
    o_ref[...] = lax.dot_general(
        onehot, zn, (((0,), (0,)), ((), ())),
        preferred_element_type=jnp.float32)


_k3 = pl.pallas_call(
    _k3_body,
    out_shape=jax.ShapeDtypeStruct((G, D), jnp.float32),
)


def kernel(x, edge_index, graph_ids, W1, b1, g1, be1, W2, b2, g2, be2):
    pad = EPAD - E
    i = jnp.arange(pad, dtype=jnp.int32)
    trash = N + (i % (NDEG - N))
    src = edge_index[0]
    dst = edge_index[1]

    edges_deg = jnp.stack([
        jnp.concatenate([src, trash]),
        jnp.concatenate([dst, trash]),
    ]).reshape(2, 16, CH_D, B)

    edges_agg = jnp.stack([
        jnp.concatenate([src, (i * 7919) % N]).reshape(NW, CH_A, B),
        jnp.concatenate([dst, N + (i % (NAGG - N))]).reshape(NW, CH_A, B),
    ], axis=2)

    zeros_1d = jnp.zeros((RPT,), jnp.float32)
    zeros_2d = jnp.zeros((B, D), jnp.float32)

    deg = _deg_kernel(edges_deg, zeros_1d)
    out_deg = deg[0, :N].reshape(N, 1)
    in_deg = deg[1, :N].reshape(N, 1)

    b1r = b1.reshape(1, D)
    g1r = g1.reshape(1, D)
    be1r = be1.reshape(1, D)
    b2r = b2.reshape(1, D)
    g2r = g2.reshape(1, D)
    be2r = be2.reshape(1, D)
    gid = graph_ids.reshape(N, 1)

    h1 = _k1(x, W1, out_deg)
    p1 = _agg_kernel(h1, edges_agg, zeros_2d)
    h2 = _k2(p1, in_deg, out_deg, b1r, g1r, be1r, W2)
    p2 = _agg_kernel(h2, edges_agg, zeros_2d)
    return _k3(p2, in_deg, b2r, g2r, be2r, gid)


# speedup vs baseline: 1.0579x; 1.0579x over previous
import functools

import jax
import jax.numpy as jnp
from jax import lax
from jax.experimental import pallas as pl
from jax.experimental.pallas import tpu as pltpu
from jax.experimental.pallas import tpu_sc as plsc

N = 10000
E = 320000
D = 128
G = 64

B = 128
NW = 32
EPAD = 323584
CH_A = EPAD // NW // B
CH_D = EPAD // 16 // B
NDEG = 10240
RPT = NDEG // 16
NAGG = 10112
RPA = NAGG // 16
IDEP = 4

_mesh = plsc.VectorSubcoreMesh(core_axis_name="c", subcore_axis_name="s")


@functools.partial(
    pl.kernel,
    out_type=jax.ShapeDtypeStruct((2, NDEG), jnp.float32),
    mesh=_mesh,
    scratch_types=[
        pltpu.VMEM((CH_D, B), jnp.int32),
        pltpu.VMEM((B,), jnp.float32),
        pltpu.VMEM((RPT,), jnp.float32),
        pltpu.VMEM_SHARED((NDEG,), jnp.float32),
        pltpu.SemaphoreType.DMA,
    ],
)
def _deg_kernel(edges_hbm, zeros_hbm, out_hbm, idx_v, ones_v, zb_v, acc_sh,
                ssem):
    c = lax.axis_index("c")
    s = lax.axis_index("s")
    pltpu.sync_copy(zeros_hbm, zb_v)
    pltpu.sync_copy(zb_v, acc_sh.at[pl.ds(s * RPT, RPT)])
    for i in range(B // 16):
        ones_v[pl.ds(i * 16, 16)] = jnp.full((16,), 1.0, jnp.float32)
    pltpu.sync_copy(edges_hbm.at[c, s], idx_v)
    plsc.subcore_barrier()
    pend = [pltpu.async_copy(ones_v, acc_sh.at[idx_v.at[j]], ssem, add=True)
            for j in range(CH_D)]
    for p in pend:
        p.wait()
    plsc.subcore_barrier()
    pltpu.sync_copy(acc_sh.at[pl.ds(s * RPT, RPT)], zb_v)
    pltpu.sync_copy(zb_v, out_hbm.at[c, pl.ds(s * RPT, RPT)])


@functools.partial(
    pl.kernel,
    out_type=jax.ShapeDtypeStruct((2, NAGG, D), jnp.float32),
    mesh=_mesh,
    scratch_types=[
        pltpu.VMEM((IDEP, 2, B), jnp.int32),
        pltpu.VMEM((B, D), jnp.float32),
        pltpu.VMEM((B, D), jnp.float32),
        pltpu.VMEM((B, D), jnp.float32),
        pltpu.VMEM_SHARED((NAGG, D), jnp.float32),
        pltpu.SemaphoreType.DMA,
        pltpu.SemaphoreType.DMA,
        pltpu.SemaphoreType.DMA,
        pltpu.SemaphoreType.DMA,
        pltpu.SemaphoreType.DMA,
        pltpu.SemaphoreType.DMA,
        pltpu.SemaphoreType.DMA,
        pltpu.SemaphoreType.DMA,
        pltpu.SemaphoreType.DMA,
    ],
)
def _agg_kernel(h_hbm, edges_hbm, zeros_hbm, out_hbm,
                idx_v, r0, r1, r2, acc_sh,
                g0, g1, g2, s0, s1, i0, i1, i2, i3):
    c = lax.axis_index("c")
    s = lax.axis_index("s")
    w = c * 16 + s
    rows = (r0, r1, r2)
    gsems = (g0, g1, g2)
    ssems = (s0, s1)
    isems = (i0, i1, i2, i3)
    gp = [None] * CH_A
    ip = [None] * CH_A
    sp = [None] * CH_A
    for k in range(min(3, CH_A)):
        ip[k] = pltpu.async_copy(edges_hbm.at[w, k], idx_v.at[k],
                                 isems[k])
    for k in range(min(2, CH_A)):
        ip[k].wait()
        gp[k] = pltpu.async_copy(h_hbm.at[idx_v.at[k, 0]], rows[k], gsems[k])
    pltpu.sync_copy(zeros_hbm, r2)
    zp = []
    for t in range(5):
        n = 128 if t < 4 else RPA - 512
        zp.append(pltpu.async_copy(r2.at[pl.ds(0, n)],
                                   acc_sh.at[pl.ds(s * RPA + t * 128, n)],
                                   s0))
    for p in zp:
        p.wait()
    plsc.subcore_barrier()
    for j in range(CH_A):
        gp[j].wait()
        sp[j] = pltpu.async_copy(rows[j % 3], acc_sh.at[idx_v.at[j % 4, 1]],
                                 ssems[j % 2], add=True)
        if j >= 1:
            sp[j - 1].wait()
        if j + 3 < CH_A:
            ip[j + 3] = pltpu.async_copy(edges_hbm.at[w, j + 3],
                                         idx_v.at[(j + 3) % 4],
                                         isems[(j + 3) % 4])
        if j + 2 < CH_A:
            if ip[j + 2] is not None:
                ip[j + 2].wait()
            gp[j + 2] = pltpu.async_copy(h_hbm.at[idx_v.at[(j + 2) % 4, 0]],
                                         rows[(j + 2) % 3],
                                         gsems[(j + 2) % 3])
    sp[CH_A - 1].wait()
    plsc.subcore_barrier()
    sz = [128, 128, 128, 128, RPA - 512]
    pin = [None] * 5
    pout = [None] * 5
    for t in range(3):
        pin[t] = pltpu.async_copy(
            acc_sh.at[pl.ds(s * RPA + t * 128, sz[t])],
            rows[t].at[pl.ds(0, sz[t])], gsems[t])
    for t in range(5):
        pin[t].wait()
        pout[t] = pltpu.async_copy(
            rows[t % 3].at[pl.ds(0, sz[t])],
            out_hbm.at[c, pl.ds(s * RPA + t * 128, sz[t])],
            ssems[t % 2])
        u = t + 3
        if u in (3, 4):
            pout[t].wait()
            pin[u] = pltpu.async_copy(
                acc_sh.at[pl.ds(s * RPA + u * 128, sz[u])],
                rows[t % 3].at[pl.ds(0, sz[u])], gsems[t % 3])
    pout[2].wait()
    pout[3].wait()
    pout[4].wait()


def _k1_body(x_ref, w_ref, deg_ref, o_ref):
    norm = lax.rsqrt(jnp.maximum(deg_ref[...], 1.0))
    o_ref[...] = jnp.dot(x_ref[...], w_ref[...],
                         preferred_element_type=jnp.float32) * norm


_k1 = pl.pallas_call(
    _k1_body,
    out_shape=jax.ShapeDtypeStruct((N, D), jnp.float32),
)


def _k2_body(p_ref, indeg_ref, outdeg_ref, b1_ref, g1_ref, be1_ref, w2_ref,
             o_ref):
    p = p_ref[0, :N, :] + p_ref[1, :N, :]
    z = p * lax.rsqrt(jnp.maximum(indeg_ref[...], 1.0)) + b1_ref[...]
    mu = jnp.mean(z, axis=0, keepdims=True)
    var = jnp.mean((z - mu) ** 2, axis=0, keepdims=True)
    zn = (z - mu) * lax.rsqrt(var + 1e-5) * g1_ref[...] + be1_ref[...]
    zn = jnp.maximum(zn, 0.0)
    nsrc = lax.rsqrt(jnp.maximum(outdeg_ref[...], 1.0))
    o_ref[...] = jnp.dot(zn, w2_ref[...],
                         preferred_element_type=jnp.float32) * nsrc


_k2 = pl.pallas_call(
    _k2_body,
    out_shape=jax.ShapeDtypeStruct((N, D), jnp.float32),
)


def _k3_body(p_ref, indeg_ref, b2_ref, g2_ref, be2_ref, gid_ref, o_ref):
    p = p_ref[0, :N, :] + p_ref[1, :N, :]
    z = p * lax.rsqrt(jnp.maximum(indeg_ref[...], 1.0)) + b2_ref[...]
    mu = jnp.mean(z, axis=0, keepdims=True)
    var = jnp.mean((z - mu) ** 2, axis=0, keepdims=True)
    zn = (z - mu) * lax.rsqrt(var + 1e-5) * g2_ref[...] + be2_ref[...]
    iota = lax.broadcasted_iota(jnp.int32, (1, G), 1)
    onehot = (gid_ref[...] == iota).astype(jnp.float32)
    o_ref[...] = lax.dot_general(
        onehot, zn, (((0,), (0,)), ((), ())),
        preferred_element_type=jnp.float32)


_k3 = pl.pallas_call(
    _k3_body,
    out_shape=jax.ShapeDtypeStruct((G, D), jnp.float32),
)


def kernel(x, edge_index, graph_ids, W1, b1, g1, be1, W2, b2, g2, be2):
    pad = EPAD - E
    i = jnp.arange(pad, dtype=jnp.int32)
    trash = N + (i % (NDEG - N))
    src = edge_index[0]
    dst = edge_index[1]

    edges_deg = jnp.stack([
        jnp.concatenate([src, trash]),
        jnp.concatenate([dst, trash]),
    ]).reshape(2, 16, CH_D, B)

    edges_agg = jnp.stack([
        jnp.concatenate([src, (i * 7919) % N]).reshape(NW, CH_A, B),
        jnp.concatenate([dst, N + (i % (NAGG - N))]).reshape(NW, CH_A, B),
    ], axis=2)

    zeros_1d = jnp.zeros((RPT,), jnp.float32)
    zeros_2d = jnp.zeros((B, D), jnp.float32)

    deg = _deg_kernel(edges_deg, zeros_1d)
    out_deg = deg[0, :N].reshape(N, 1)
    in_deg = deg[1, :N].reshape(N, 1)

    b1r = b1.reshape(1, D)
    g1r = g1.reshape(1, D)
    be1r = be1.reshape(1, D)
    b2r = b2.reshape(1, D)
    g2r = g2.reshape(1, D)
    be2r = be2.reshape(1, D)
    gid = graph_ids.reshape(N, 1)

    h1 = _k1(x, W1, out_deg)
    p1 = _agg_kernel(h1, edges_agg, zeros_2d)
    h2 = _k2(p1, in_deg, out_deg, b1r, g1r, be1r, W2)
    p2 = _agg_kernel(h2, edges_agg, zeros_2d)
    return _k3(p2, in_deg, b2r, g2r, be2r, gid)
